# Initial kernel scaffold; baseline (speedup 1.0000x reference)
import jax
import jax.numpy as jnp
from jax.experimental import pallas as pl


def kernel(x, edge_index, W1, W2, W3):
    raise NotImplementedError("write your pallas kernel here")


# speedup vs baseline: 8.1358x; 8.1358x over previous
import functools

import jax
import jax.numpy as jnp
from jax import lax
from jax.experimental import pallas as pl
from jax.experimental.pallas import tpu as pltpu
from jax.experimental.pallas import tpu_sc as plsc

N = 10000
D = 128
H1 = 64
H2 = 32
E = 320000

NCORES = 2
NSUB = 16
NW = NCORES * NSUB
CHUNK = 128
NCH = (E // NW + CHUNK - 1) // CHUNK
EPAD = NW * NCH * CHUNK
NP = 10112
RPT = NP // NSUB


def _spmm_sc(src_c, dst_c, h, zeros):
    F = h.shape[1]
    mesh = plsc.VectorSubcoreMesh(core_axis_name="c", subcore_axis_name="s")

    @functools.partial(
        pl.kernel,
        out_type=jax.ShapeDtypeStruct((NCORES, NP, F), jnp.float32),
        mesh=mesh,
        scratch_types=[
            pltpu.VMEM((NCH, CHUNK), jnp.int32),
            pltpu.VMEM((NCH, CHUNK), jnp.int32),
            pltpu.VMEM((CHUNK, F), jnp.float32),
            pltpu.VMEM_SHARED((NP, F), jnp.float32),
        ],
        compiler_params=pltpu.CompilerParams(use_tc_tiling_on_sc=False),
    )
    def spmm(src_hbm, dst_hbm, h_hbm, z_hbm, out_hbm, src_v, dst_v, rows_v, acc):
        c = lax.axis_index("c")
        s = lax.axis_index("s")
        w = c * NSUB + s
        row0 = s * RPT
        pltpu.sync_copy(z_hbm.at[pl.ds(row0, RPT)], acc.at[pl.ds(row0, RPT)])
        pltpu.sync_copy(src_hbm.at[w], src_v)
        pltpu.sync_copy(dst_hbm.at[w], dst_v)
        plsc.subcore_barrier()

        @pl.loop(0, NCH)
        def _(j):
            pltpu.sync_copy(h_hbm.at[src_v.at[j]], rows_v)
            pltpu.sync_copy(rows_v, acc.at[dst_v.at[j]], add=True)

        plsc.subcore_barrier()
        pltpu.sync_copy(acc.at[pl.ds(row0, RPT)],
                        out_hbm.at[c, pl.ds(row0, RPT)])

    return spmm(src_c, dst_c, h, zeros)


def _mm_body(x_ref, w_ref, o_ref):
    o_ref[...] = jnp.dot(x_ref[...], w_ref[...],
                         preferred_element_type=jnp.float32)


def _mid_body(p0_ref, p1_ref, w_ref, o_ref):
    h = jnp.maximum(p0_ref[...] + p1_ref[...], 0.0)
    o_ref[...] = jnp.dot(h, w_ref[...], preferred_element_type=jnp.float32)


def _reparam_body(q0_ref, q1_ref, eps_ref, z_ref, mu_ref, lv_ref):
    mu = q0_ref[:, :H2] + q1_ref[:, :H2]
    lv = q0_ref[:, H2:] + q1_ref[:, H2:]
    mu_ref[...] = mu
    lv_ref[...] = lv
    z_ref[...] = eps_ref[...] * jnp.exp(lv) + mu


def _outer_body(zi_ref, zj_ref, o_ref):
    o_ref[...] = lax.dot_general(zi_ref[...], zj_ref[...],
                                 (((1,), (1,)), ((), ())),
                                 preferred_element_type=jnp.float32)


_RB = 2000
_OB = 400


def kernel(x, edge_index, W1, W2, W3):
    pad = EPAD - E
    src_c = jnp.concatenate(
        [edge_index[0], jnp.full((pad,), N, jnp.int32)]).reshape(NW, NCH, CHUNK)
    dst_c = jnp.concatenate(
        [edge_index[1], jnp.full((pad,), N, jnp.int32)]).reshape(NW, NCH, CHUNK)
    zeros64 = jnp.zeros((NP, H1), jnp.float32)
    eps = jax.random.normal(jax.random.key(42), (N, H2), dtype=jnp.float32)
    W23 = jnp.concatenate([W2, W3], axis=1)

    h0 = pl.pallas_call(
        _mm_body,
        grid=(N // _RB,),
        in_specs=[pl.BlockSpec((_RB, D), lambda i: (i, 0)),
                  pl.BlockSpec((D, H1), lambda i: (0, 0))],
        out_specs=pl.BlockSpec((_RB, H1), lambda i: (i, 0)),
        out_shape=jax.ShapeDtypeStruct((N, H1), jnp.float32),
    )(x, W1)

    h0p = jnp.concatenate([h0, jnp.zeros((NP - N, H1), jnp.float32)], axis=0)
    parts1 = _spmm_sc(src_c, dst_c, h0p, zeros64)

    h23 = pl.pallas_call(
        _mid_body,
        grid=(N // _RB,),
        in_specs=[pl.BlockSpec((_RB, H1), lambda i: (i, 0)),
                  pl.BlockSpec((_RB, H1), lambda i: (i, 0)),
                  pl.BlockSpec((H1, H1), lambda i: (0, 0))],
        out_specs=pl.BlockSpec((_RB, H1), lambda i: (i, 0)),
        out_shape=jax.ShapeDtypeStruct((N, H1), jnp.float32),
    )(parts1[0, :N], parts1[1, :N], W23)

    h23p = jnp.concatenate([h23, jnp.zeros((NP - N, H1), jnp.float32)], axis=0)
    parts2 = _spmm_sc(src_c, dst_c, h23p, zeros64)

    z, mu, logvar = pl.pallas_call(
        _reparam_body,
        grid=(N // _RB,),
        in_specs=[pl.BlockSpec((_RB, H1), lambda i: (i, 0)),
                  pl.BlockSpec((_RB, H1), lambda i: (i, 0)),
                  pl.BlockSpec((_RB, H2), lambda i: (i, 0))],
        out_specs=[pl.BlockSpec((_RB, H2), lambda i: (i, 0)),
                   pl.BlockSpec((_RB, H2), lambda i: (i, 0)),
                   pl.BlockSpec((_RB, H2), lambda i: (i, 0))],
        out_shape=[jax.ShapeDtypeStruct((N, H2), jnp.float32),
                   jax.ShapeDtypeStruct((N, H2), jnp.float32),
                   jax.ShapeDtypeStruct((N, H2), jnp.float32)],
    )(parts2[0, :N], parts2[1, :N], eps)

    pred_adj = pl.pallas_call(
        _outer_body,
        grid=(N // _OB,),
        in_specs=[pl.BlockSpec((_OB, H2), lambda i: (i, 0)),
                  pl.BlockSpec((N, H2), lambda i: (0, 0))],
        out_specs=pl.BlockSpec((_OB, N), lambda i: (i, 0)),
        out_shape=jax.ShapeDtypeStruct((N, N), jnp.float32),
    )(z, z)

    return (pred_adj, mu, logvar)
